# baseline (speedup 1.0000x reference)
import functools

import jax
import jax.numpy as jnp
from jax import lax
from jax.experimental import pallas as pl
from jax.experimental.pallas import tpu as pltpu
from jax.experimental.pallas import tpu_sc as plsc

_B = 2048
_IN = 512
_OUT = 32
_T = 32
_DEPTH = 6
_S = 31
_L = 32
_F = 16
_TL = _T * _L
_K = _TL * _F

_BT = 256
_BTH = 128
_NT = _B // _BT
_NLC = 128
_CK = _NLC * _F
_FP = _F + 1
_NCHUNK = _TL // _NLC

_NW = 32
_KPW = _K // _NW
_RC = 32
_NCH_SC = _KPW // _RC


def _smooth_step(t):
    tc = jnp.clip(t, -0.5, 0.5)
    return tc * (1.5 - 2.0 * tc * tc) + 0.5


def _sc_gather_body(xu_hbm, ids_hbm, out_hbm,
                    idx_v, rows0, rows1, gs0, gs1, ss0, ss1):
    wid = lax.axis_index("s") * 2 + lax.axis_index("c")
    base = wid * _KPW
    pltpu.sync_copy(ids_hbm.at[pl.ds(base, _KPW)], idx_v)
    bufs = (rows0, rows1)
    gsems = (gs0, gs1)
    ssems = (ss0, ss1)

    def _gather(j):
        return pltpu.async_copy(
            xu_hbm.at[idx_v.at[pl.ds(j * _RC, _RC)]], bufs[j % 2],
            gsems[j % 2])

    gd = [_gather(0), _gather(1)]
    for j in range(_NCH_SC):
        b = j % 2
        gd[b].wait()
        sd = pltpu.async_copy(
            bufs[b], out_hbm.at[pl.ds(base + j * _RC, _RC)], ssems[b])
        sd.wait()
        if j + 2 < _NCH_SC:
            gd[b] = _gather(j + 2)


def _sc_gather(xu, ids_flat):
    mesh = plsc.VectorSubcoreMesh(core_axis_name="c", subcore_axis_name="s")
    return pl.kernel(
        _sc_gather_body,
        mesh=mesh,
        out_type=jax.ShapeDtypeStruct((_K, _B // 2), jnp.uint32),
        scratch_types=[
            pltpu.VMEM((_KPW,), jnp.int32),
            pltpu.VMEM((_RC, _B // 2), jnp.uint32),
            pltpu.VMEM((_RC, _B // 2), jnp.uint32),
            pltpu.SemaphoreType.DMA,
            pltpu.SemaphoreType.DMA,
            pltpu.SemaphoreType.DMA,
            pltpu.SemaphoreType.DMA,
        ],
    )(xu, ids_flat)


def _prepack_kernel(xT_ref, xu_ref):
    xb = xT_ref[...].astype(jnp.bfloat16)
    lo = lax.convert_element_type(
        lax.bitcast_convert_type(xb[:, :_BTH], jnp.uint16), jnp.uint32)
    hi = lax.convert_element_type(
        lax.bitcast_convert_type(xb[:, _BTH:], jnp.uint16), jnp.uint32)
    xu_ref[...] = lo | lax.shift_left(hi, jnp.uint32(16))


def _route_kernel(xT_ref, slopes_ref, bias_ref, aT_ref):
    t = jax.lax.dot_general(
        slopes_ref[...], xT_ref[...], (((1,), (0,)), ((), ())),
        preferred_element_type=jnp.float32)
    s = _smooth_step(t + bias_ref[...])
    s3 = s.reshape(_T, _S, _BT)
    aT = None
    for d in range(_DEPTH - 1):
        nb, ne = 2 ** d - 1, 2 ** (d + 1) - 1
        lvl = s3[:, nb:ne, :].reshape(_T, ne - nb, 1, _BT)
        rep = jnp.broadcast_to(lvl, (_T, ne - nb, _L // (ne - nb), _BT))
        rep = rep.reshape(_T, _L, _BT)
        lidx = jax.lax.broadcasted_iota(jnp.int32, (1, _L, 1), 1)
        bit = ((lidx >> (_DEPTH - 2 - d)) & 1).astype(jnp.float32)
        f = (2.0 * bit - 1.0) * rep + (1.0 - bit)
        aT = f if aT is None else aT * f
    aT_ref[...] = aT.reshape(_TL, 1, 2, _BTH)


def _wtrans_kernel(lc_ref, w2a_ref):
    w2a_ref[...] = jnp.transpose(lc_ref[...], (0, 2, 1))


def _main_kernel(xf_ref, a_ref, w2a_ref, out_ref):
    acc = [None, None]
    for c in range(_NCHUNK):
        w = xf_ref[pl.ds(c * _CK, _CK), :]
        xs = (jax.lax.bitcast_convert_type(
                  jax.lax.shift_left(w, jnp.uint32(16)), jnp.float32),
              jax.lax.bitcast_convert_type(
                  w & jnp.uint32(0xFFFF0000), jnp.float32))
        a4 = a_ref[pl.ds(c * _NLC, _NLC)]
        w2a = w2a_ref[pl.ds(c * _NLC, _NLC)].reshape(_NLC * _FP, _OUT)
        for h in range(2):
            a_h = a4[:, 0, h, :].reshape(_NLC, 1, _BTH)
            y = jnp.broadcast_to(
                a_h, (_NLC, _F, _BTH)).reshape(_CK, _BTH) * xs[h]
            y_aug = jnp.concatenate(
                [y.reshape(_NLC, _F, _BTH), a_h], axis=1)
            d = jax.lax.dot_general(
                y_aug.reshape(_NLC * _FP, _BTH), w2a,
                (((0,), (0,)), ((), ())),
                preferred_element_type=jnp.float32)
            acc[h] = d if acc[h] is None else acc[h] + d
    out_ref[0, 0] = acc[0]
    out_ref[0, 1] = acc[1]


@jax.jit
def kernel(x, split_coefs, leaves_feat_ids, leaves_coefs):
    xT = x.T
    slopes = split_coefs[:, :, :-1].reshape(_T * _S, _IN)
    bias = split_coefs[:, :, -1].reshape(_T * _S, 1)
    ids_flat = leaves_feat_ids.astype(jnp.int32).reshape(_K)
    lc = leaves_coefs.reshape(_TL, _OUT, _FP)

    xu = pl.pallas_call(
        _prepack_kernel,
        grid=(_NT,),
        in_specs=[pl.BlockSpec((_IN, _BT), lambda i: (0, i))],
        out_specs=pl.BlockSpec((_IN, _BTH), lambda i: (0, i)),
        out_shape=jax.ShapeDtypeStruct((_IN, _B // 2), jnp.uint32),
    )(xT)

    xfT = _sc_gather(xu, ids_flat)

    a4 = pl.pallas_call(
        _route_kernel,
        grid=(_NT,),
        in_specs=[
            pl.BlockSpec((_IN, _BT), lambda i: (0, i)),
            pl.BlockSpec((_T * _S, _IN), lambda i: (0, 0)),
            pl.BlockSpec((_T * _S, 1), lambda i: (0, 0)),
        ],
        out_specs=pl.BlockSpec((_TL, 1, 2, _BTH), lambda i: (0, i, 0, 0)),
        out_shape=jax.ShapeDtypeStruct((_TL, _NT, 2, _BTH), jnp.float32),
    )(xT, slopes, bias)

    w2a = pl.pallas_call(
        _wtrans_kernel,
        grid=(_NCHUNK,),
        in_specs=[pl.BlockSpec((_NLC, _OUT, _FP), lambda c: (c, 0, 0))],
        out_specs=pl.BlockSpec((_NLC, _FP, _OUT), lambda c: (c, 0, 0)),
        out_shape=jax.ShapeDtypeStruct((_TL, _FP, _OUT), jnp.float32),
    )(lc)

    out = pl.pallas_call(
        _main_kernel,
        grid=(_NT,),
        in_specs=[
            pl.BlockSpec((_K, _BTH), lambda i: (0, i)),
            pl.BlockSpec((_TL, 1, 2, _BTH), lambda i: (0, i, 0, 0)),
            pl.BlockSpec((_TL, _FP, _OUT), lambda i: (0, 0, 0)),
        ],
        out_specs=pl.BlockSpec((1, 2, _BTH, _OUT), lambda i: (i, 0, 0, 0)),
        out_shape=jax.ShapeDtypeStruct((_NT, 2, _BTH, _OUT), jnp.float32),
    )(xfT, a4, w2a)
    return out.reshape(_B, _OUT)


# speedup vs baseline: 1.5327x; 1.5327x over previous
import functools

import jax
import jax.numpy as jnp
from jax import lax
from jax.experimental import pallas as pl
from jax.experimental.pallas import tpu as pltpu
from jax.experimental.pallas import tpu_sc as plsc

_B = 2048
_IN = 512
_OUT = 32
_T = 32
_DEPTH = 6
_S = 31
_L = 32
_F = 16
_TL = _T * _L
_K = _TL * _F

_BT = 256
_BTH = 128
_NT = _B // _BT
_NLC = 128
_CK = _NLC * _F
_FP = _F + 1
_NCHUNK = _TL // _NLC

_NW = 32
_KPW = _K // _NW
_RC = 32
_NCH_SC = _KPW // _RC


def _smooth_step(t):
    tc = jnp.clip(t, -0.5, 0.5)
    return tc * (1.5 - 2.0 * tc * tc) + 0.5


def _sc_gather_body(xu_hbm, ids_hbm, out_hbm,
                    idx_v, rows0, rows1, gs0, gs1, ss0, ss1):
    wid = lax.axis_index("s") * 2 + lax.axis_index("c")
    base = wid * _KPW
    pltpu.sync_copy(ids_hbm.at[pl.ds(base, _KPW)], idx_v)
    bufs = (rows0, rows1)
    gsems = (gs0, gs1)
    ssems = (ss0, ss1)

    def _gather(j):
        return pltpu.async_copy(
            xu_hbm.at[idx_v.at[pl.ds(j * _RC, _RC)]], bufs[j % 2],
            gsems[j % 2])

    gd = [_gather(0), _gather(1)]
    for j in range(_NCH_SC):
        b = j % 2
        gd[b].wait()
        sd = pltpu.async_copy(
            bufs[b], out_hbm.at[pl.ds(base + j * _RC, _RC)], ssems[b])
        sd.wait()
        if j + 2 < _NCH_SC:
            gd[b] = _gather(j + 2)


def _sc_gather(xu, ids_flat):
    mesh = plsc.VectorSubcoreMesh(core_axis_name="c", subcore_axis_name="s")
    return pl.kernel(
        _sc_gather_body,
        mesh=mesh,
        out_type=jax.ShapeDtypeStruct((_K, _B // 2), jnp.uint32),
        scratch_types=[
            pltpu.VMEM((_KPW,), jnp.int32),
            pltpu.VMEM((_RC, _B // 2), jnp.uint32),
            pltpu.VMEM((_RC, _B // 2), jnp.uint32),
            pltpu.SemaphoreType.DMA,
            pltpu.SemaphoreType.DMA,
            pltpu.SemaphoreType.DMA,
            pltpu.SemaphoreType.DMA,
        ],
    )(xu, ids_flat)


def _prepack_kernel(xT_ref, xu_ref):
    xb = xT_ref[...].astype(jnp.bfloat16)
    lo = lax.convert_element_type(
        lax.bitcast_convert_type(xb[:, :_BTH], jnp.uint16), jnp.uint32)
    hi = lax.convert_element_type(
        lax.bitcast_convert_type(xb[:, _BTH:], jnp.uint16), jnp.uint32)
    xu_ref[...] = lo | lax.shift_left(hi, jnp.uint32(16))


def _route_kernel(xT_ref, slopes_ref, bias_ref, aT_ref):
    t = jax.lax.dot_general(
        slopes_ref[...], xT_ref[...], (((1,), (0,)), ((), ())),
        preferred_element_type=jnp.float32)
    s = _smooth_step(t + bias_ref[...])
    s3 = s.reshape(_T, _S, _BT)
    aT = None
    for d in range(_DEPTH - 1):
        nb, ne = 2 ** d - 1, 2 ** (d + 1) - 1
        lvl = s3[:, nb:ne, :].reshape(_T, ne - nb, 1, _BT)
        rep = jnp.broadcast_to(lvl, (_T, ne - nb, _L // (ne - nb), _BT))
        rep = rep.reshape(_T, _L, _BT)
        lidx = jax.lax.broadcasted_iota(jnp.int32, (1, _L, 1), 1)
        bit = ((lidx >> (_DEPTH - 2 - d)) & 1).astype(jnp.float32)
        f = (2.0 * bit - 1.0) * rep + (1.0 - bit)
        aT = f if aT is None else aT * f
    aT_ref[...] = aT.reshape(_TL, 1, 2, _BTH)


def _wtrans_kernel(lc_ref, w2f_ref, wb_ref):
    lc = lc_ref[...]
    w2f_ref[...] = jnp.transpose(
        lc[:, :, :_F], (0, 2, 1)).reshape(_CK, _OUT)
    wb_ref[...] = lc[:, :, _F]


def _main_kernel(xf_ref, a_ref, w2f_ref, wb_ref, out_ref):
    c = pl.program_id(1)
    w = xf_ref[...]
    xs = (jax.lax.bitcast_convert_type(
              jax.lax.shift_left(w, jnp.uint32(16)), jnp.float32),
          jax.lax.bitcast_convert_type(
              w & jnp.uint32(0xFFFF0000), jnp.float32))
    w2f = w2f_ref[...]
    wb = wb_ref[...]
    a4 = a_ref[...]
    for h in range(2):
        a_h = a4[:, 0, h, :]
        y = jnp.broadcast_to(
            a_h.reshape(_NLC, 1, _BTH),
            (_NLC, _F, _BTH)).reshape(_CK, _BTH) * xs[h]
        d = jax.lax.dot_general(
            y, w2f, (((0,), (0,)), ((), ())),
            preferred_element_type=jnp.float32)
        d = d + jax.lax.dot_general(
            a_h, wb, (((0,), (0,)), ((), ())),
            preferred_element_type=jnp.float32)

        @pl.when(c == 0)
        def _init(h=h, d=d):
            out_ref[0, h] = d

        @pl.when(c > 0)
        def _acc(h=h, d=d):
            out_ref[0, h] += d


@jax.jit
def kernel(x, split_coefs, leaves_feat_ids, leaves_coefs):
    xT = x.T
    slopes = split_coefs[:, :, :-1].reshape(_T * _S, _IN)
    bias = split_coefs[:, :, -1].reshape(_T * _S, 1)
    ids_flat = leaves_feat_ids.astype(jnp.int32).reshape(_K)
    lc = leaves_coefs.reshape(_TL, _OUT, _FP)

    xu = pl.pallas_call(
        _prepack_kernel,
        grid=(_NT,),
        in_specs=[pl.BlockSpec((_IN, _BT), lambda i: (0, i))],
        out_specs=pl.BlockSpec((_IN, _BTH), lambda i: (0, i)),
        out_shape=jax.ShapeDtypeStruct((_IN, _B // 2), jnp.uint32),
    )(xT)

    xfT = _sc_gather(xu, ids_flat)

    a4 = pl.pallas_call(
        _route_kernel,
        grid=(_NT,),
        in_specs=[
            pl.BlockSpec((_IN, _BT), lambda i: (0, i)),
            pl.BlockSpec((_T * _S, _IN), lambda i: (0, 0)),
            pl.BlockSpec((_T * _S, 1), lambda i: (0, 0)),
        ],
        out_specs=pl.BlockSpec((_TL, 1, 2, _BTH), lambda i: (0, i, 0, 0)),
        out_shape=jax.ShapeDtypeStruct((_TL, _NT, 2, _BTH), jnp.float32),
    )(xT, slopes, bias)

    w2f, wb = pl.pallas_call(
        _wtrans_kernel,
        grid=(_NCHUNK,),
        in_specs=[pl.BlockSpec((_NLC, _OUT, _FP), lambda c: (c, 0, 0))],
        out_specs=[
            pl.BlockSpec((_CK, _OUT), lambda c: (c, 0)),
            pl.BlockSpec((_NLC, _OUT), lambda c: (c, 0)),
        ],
        out_shape=[
            jax.ShapeDtypeStruct((_K, _OUT), jnp.float32),
            jax.ShapeDtypeStruct((_TL, _OUT), jnp.float32),
        ],
    )(lc)

    out = pl.pallas_call(
        _main_kernel,
        grid=(_NT, _NCHUNK),
        in_specs=[
            pl.BlockSpec((_CK, _BTH), lambda i, c: (c, i)),
            pl.BlockSpec((_NLC, 1, 2, _BTH), lambda i, c: (c, i, 0, 0)),
            pl.BlockSpec((_CK, _OUT), lambda i, c: (c, 0)),
            pl.BlockSpec((_NLC, _OUT), lambda i, c: (c, 0)),
        ],
        out_specs=pl.BlockSpec((1, 2, _BTH, _OUT), lambda i, c: (i, 0, 0, 0)),
        out_shape=jax.ShapeDtypeStruct((_NT, 2, _BTH, _OUT), jnp.float32),
    )(xfT, a4, w2f, wb)
    return out.reshape(_B, _OUT)


# speedup vs baseline: 1.5420x; 1.0061x over previous
import functools

import jax
import jax.numpy as jnp
from jax import lax
from jax.experimental import pallas as pl
from jax.experimental.pallas import tpu as pltpu
from jax.experimental.pallas import tpu_sc as plsc

_B = 2048
_IN = 512
_OUT = 32
_T = 32
_DEPTH = 6
_S = 31
_L = 32
_F = 16
_TL = _T * _L
_K = _TL * _F

_BT = 256
_BTH = 128
_NT = _B // _BT
_NLC = 128
_CK = _NLC * _F
_FP = _F + 1
_NCHUNK = _TL // _NLC

_NW = 32
_KH = _K // 2
_KPW = _KH // _NW
_RC = 32
_NCH_SC = _KPW // _RC


def _smooth_step(t):
    tc = jnp.clip(t, -0.5, 0.5)
    return tc * (1.5 - 2.0 * tc * tc) + 0.5


def _sc_gather_body(xu_hbm, ids_hbm, out_hbm,
                    idx_v, rows0, rows1, gs0, gs1, ss0, ss1):
    wid = lax.axis_index("s") * 2 + lax.axis_index("c")
    base = wid * _KPW
    pltpu.sync_copy(ids_hbm.at[pl.ds(base, _KPW)], idx_v)
    bufs = (rows0, rows1)
    gsems = (gs0, gs1)
    ssems = (ss0, ss1)

    def _gather(j):
        return pltpu.async_copy(
            xu_hbm.at[idx_v.at[pl.ds(j * _RC, _RC)]], bufs[j % 2],
            gsems[j % 2])

    gd = [_gather(0), _gather(1)]
    for j in range(_NCH_SC):
        b = j % 2
        gd[b].wait()
        sd = pltpu.async_copy(
            bufs[b], out_hbm.at[pl.ds(base + j * _RC, _RC)], ssems[b])
        sd.wait()
        if j + 2 < _NCH_SC:
            gd[b] = _gather(j + 2)


def _sc_gather(xu, ids_flat):
    mesh = plsc.VectorSubcoreMesh(core_axis_name="c", subcore_axis_name="s")
    return pl.kernel(
        _sc_gather_body,
        mesh=mesh,
        out_type=jax.ShapeDtypeStruct((_KH, _B // 2), jnp.uint32),
        scratch_types=[
            pltpu.VMEM((_KPW,), jnp.int32),
            pltpu.VMEM((_RC, _B // 2), jnp.uint32),
            pltpu.VMEM((_RC, _B // 2), jnp.uint32),
            pltpu.SemaphoreType.DMA,
            pltpu.SemaphoreType.DMA,
            pltpu.SemaphoreType.DMA,
            pltpu.SemaphoreType.DMA,
        ],
    )(xu, ids_flat)


def _prepack_kernel(xT_ref, xu_ref):
    xb = xT_ref[...].astype(jnp.bfloat16)
    lo = lax.convert_element_type(
        lax.bitcast_convert_type(xb[:, :_BTH], jnp.uint16), jnp.uint32)
    hi = lax.convert_element_type(
        lax.bitcast_convert_type(xb[:, _BTH:], jnp.uint16), jnp.uint32)
    xu_ref[...] = lo | lax.shift_left(hi, jnp.uint32(16))


def _route_kernel(xT_ref, slopes_ref, bias_ref, aT_ref):
    t = jax.lax.dot_general(
        slopes_ref[...], xT_ref[...], (((1,), (0,)), ((), ())),
        preferred_element_type=jnp.float32)
    s = _smooth_step(t + bias_ref[...])
    s3 = s.reshape(_T, _S, _BT)
    aT = None
    for d in range(_DEPTH - 1):
        nb, ne = 2 ** d - 1, 2 ** (d + 1) - 1
        lvl = s3[:, nb:ne, :].reshape(_T, ne - nb, 1, _BT)
        rep = jnp.broadcast_to(lvl, (_T, ne - nb, _L // (ne - nb), _BT))
        rep = rep.reshape(_T, _L, _BT)
        lidx = jax.lax.broadcasted_iota(jnp.int32, (1, _L, 1), 1)
        bit = ((lidx >> (_DEPTH - 2 - d)) & 1).astype(jnp.float32)
        f = (2.0 * bit - 1.0) * rep + (1.0 - bit)
        aT = f if aT is None else aT * f
    aT_ref[...] = aT.reshape(_TL, 1, 2, _BTH)


def _wtrans_kernel(lc_ref, w2f_ref, wb_ref):
    lc = lc_ref[...]
    w2f_ref[...] = jnp.transpose(
        lc[:, :, :_F], (0, 2, 1)).reshape(_CK, _OUT)
    wb_ref[...] = lc[:, :, _F]


def _main_kernel(xf_ref, a_ref, w2f_ref, wb_ref, prev_ref, out_ref):
    c = pl.program_id(1)
    w = xf_ref[...]
    xs = (jax.lax.bitcast_convert_type(
              jax.lax.shift_left(w, jnp.uint32(16)), jnp.float32),
          jax.lax.bitcast_convert_type(
              w & jnp.uint32(0xFFFF0000), jnp.float32))
    w2f = w2f_ref[...]
    wb = wb_ref[...]
    a4 = a_ref[...]
    for h in range(2):
        a_h = a4[:, 0, h, :]
        y = jnp.broadcast_to(
            a_h.reshape(_NLC, 1, _BTH),
            (_NLC, _F, _BTH)).reshape(_CK, _BTH) * xs[h]
        d = jax.lax.dot_general(
            y, w2f, (((0,), (0,)), ((), ())),
            preferred_element_type=jnp.float32)
        d = d + jax.lax.dot_general(
            a_h, wb, (((0,), (0,)), ((), ())),
            preferred_element_type=jnp.float32)

        @pl.when(c == 0)
        def _init(h=h, d=d):
            out_ref[0, h] = prev_ref[0, h] + d

        @pl.when(c > 0)
        def _acc(h=h, d=d):
            out_ref[0, h] += d


@jax.jit
def kernel(x, split_coefs, leaves_feat_ids, leaves_coefs):
    xT = x.T
    slopes = split_coefs[:, :, :-1].reshape(_T * _S, _IN)
    bias = split_coefs[:, :, -1].reshape(_T * _S, 1)
    ids_flat = leaves_feat_ids.astype(jnp.int32).reshape(_K)
    lc = leaves_coefs.reshape(_TL, _OUT, _FP)

    xu = pl.pallas_call(
        _prepack_kernel,
        grid=(_NT,),
        in_specs=[pl.BlockSpec((_IN, _BT), lambda i: (0, i))],
        out_specs=pl.BlockSpec((_IN, _BTH), lambda i: (0, i)),
        out_shape=jax.ShapeDtypeStruct((_IN, _B // 2), jnp.uint32),
    )(xT)

    xf0 = _sc_gather(xu, ids_flat[:_KH])
    xf1 = _sc_gather(xu, ids_flat[_KH:])

    a4 = pl.pallas_call(
        _route_kernel,
        grid=(_NT,),
        in_specs=[
            pl.BlockSpec((_IN, _BT), lambda i: (0, i)),
            pl.BlockSpec((_T * _S, _IN), lambda i: (0, 0)),
            pl.BlockSpec((_T * _S, 1), lambda i: (0, 0)),
        ],
        out_specs=pl.BlockSpec((_TL, 1, 2, _BTH), lambda i: (0, i, 0, 0)),
        out_shape=jax.ShapeDtypeStruct((_TL, _NT, 2, _BTH), jnp.float32),
    )(xT, slopes, bias)

    w2f, wb = pl.pallas_call(
        _wtrans_kernel,
        grid=(_NCHUNK,),
        in_specs=[pl.BlockSpec((_NLC, _OUT, _FP), lambda c: (c, 0, 0))],
        out_specs=[
            pl.BlockSpec((_CK, _OUT), lambda c: (c, 0)),
            pl.BlockSpec((_NLC, _OUT), lambda c: (c, 0)),
        ],
        out_shape=[
            jax.ShapeDtypeStruct((_K, _OUT), jnp.float32),
            jax.ShapeDtypeStruct((_TL, _OUT), jnp.float32),
        ],
    )(lc)

    nch_h = _NCHUNK // 2

    def _main_call(xf_half, prev, coff):
        return pl.pallas_call(
            _main_kernel,
            grid=(_NT, nch_h),
            in_specs=[
                pl.BlockSpec((_CK, _BTH), lambda i, c: (c, i)),
                pl.BlockSpec((_NLC, 1, 2, _BTH),
                             lambda i, c: (c + coff, i, 0, 0)),
                pl.BlockSpec((_CK, _OUT), lambda i, c: (c + coff, 0)),
                pl.BlockSpec((_NLC, _OUT), lambda i, c: (c + coff, 0)),
                pl.BlockSpec((1, 2, _BTH, _OUT),
                             lambda i, c: (i, 0, 0, 0)),
            ],
            out_specs=pl.BlockSpec((1, 2, _BTH, _OUT),
                                   lambda i, c: (i, 0, 0, 0)),
            out_shape=jax.ShapeDtypeStruct((_NT, 2, _BTH, _OUT),
                                           jnp.float32),
        )(xf_half, a4, w2f, wb, prev)

    zeros = jnp.zeros((_NT, 2, _BTH, _OUT), jnp.float32)
    out = _main_call(xf1, _main_call(xf0, zeros, 0), nch_h)
    return out.reshape(_B, _OUT)
